# baseline (speedup 1.0000x reference)
import functools

import jax
import jax.numpy as jnp
from jax import lax
from jax.experimental import pallas as pl
from jax.experimental.pallas import tpu as pltpu
from jax.experimental.pallas import tpu_sc as plsc

B = 16384
F = 26
V = 100000
D = 16
H1 = 128
H2 = 64

NW = 32
FP = 32
S = B * FP
RPW = S // NW
NCH = 8
CH = RPW // NCH

VC = 6400
NJ = -(-V // VC)
VP = NJ * VC
NB = -(-F // 8)


def _tc_relayout(tables):
  tab_t = jnp.transpose(tables, (0, 2, 1)).reshape(F * D, V)

  def body(i_ref, o_ref):
    o_ref[...] = i_ref[...].T

  lin128 = pl.pallas_call(
      body,
      grid=(NB, NJ),
      in_specs=[pl.BlockSpec((128, VC), lambda nb, j: (nb, j))],
      out_specs=pl.BlockSpec((VC, 128), lambda nb, j: (nb * NJ + j, 0)),
      out_shape=jax.ShapeDtypeStruct((NB * VP, 128), jnp.float32),
  )(tab_t)
  return lin128.reshape(NB * VP * 8, D)


def _sc_gather(tab_flat, idx3):
  mesh = plsc.VectorSubcoreMesh(core_axis_name="c", subcore_axis_name="s")

  @functools.partial(
      pl.kernel,
      out_type=jax.ShapeDtypeStruct((S, D), jnp.float32),
      mesh=mesh,
      compiler_params=pltpu.CompilerParams(use_tc_tiling_on_sc=False),
      scratch_types=[
          pltpu.VMEM((NCH, CH), jnp.int32),
          pltpu.VMEM((2, CH, D), jnp.float32),
          pltpu.SemaphoreType.DMA,
          pltpu.SemaphoreType.DMA,
          pltpu.SemaphoreType.DMA,
          pltpu.SemaphoreType.DMA,
      ],
  )
  def body(tab_hbm, idx_hbm, out_hbm, idx_v, rows_v, g0, g1, o0, o1):
    num_s = lax.axis_size("s")
    wid = lax.axis_index("c") * num_s + lax.axis_index("s")
    base = wid * RPW
    gsem = (g0, g1)
    osem = (o0, o1)
    pltpu.sync_copy(idx_hbm.at[wid], idx_v)

    gathers = [None] * NCH
    outs = [None] * NCH
    gathers[0] = pltpu.async_copy(tab_hbm.at[idx_v.at[0]], rows_v.at[0],
                                  gsem[0])
    for i in range(NCH):
      p = i % 2
      if i + 1 < NCH:
        if i >= 1:
          outs[i - 1].wait()
        gathers[i + 1] = pltpu.async_copy(
            tab_hbm.at[idx_v.at[i + 1]], rows_v.at[1 - p], gsem[1 - p])
      gathers[i].wait()
      outs[i] = pltpu.async_copy(
          rows_v.at[p], out_hbm.at[pl.ds(base + i * CH, CH)], osem[p])
    outs[NCH - 2].wait()
    outs[NCH - 1].wait()

  return body(tab_flat, idx3)


def _mlp(emb128, W1p, b1, W2, b2):
  BB = 2048
  GB = BB * FP * D // 128

  def body(e_ref, w1_ref, b1_ref, w2_ref, b2_ref, o_ref):
    a = e_ref[...].reshape(BB // 8, 4, 8, 128)
    h = b1_ref[...]
    for ct in range(4):
      xct = a[:, ct, :, :].reshape(BB, 128)
      h = h + jnp.dot(xct, w1_ref[ct * 128:(ct + 1) * 128, :],
                      preferred_element_type=jnp.float32)
    h = jnp.where(h >= 0, h, 0.01 * h)
    h = jnp.dot(h, w2_ref[...], preferred_element_type=jnp.float32)
    h = h + b2_ref[...]
    o_ref[...] = jnp.where(h >= 0, h, 0.01 * h)

  return pl.pallas_call(
      body,
      grid=(B // BB,),
      in_specs=[
          pl.BlockSpec((GB, 128), lambda i: (i, 0)),
          pl.BlockSpec((FP * D, H1), lambda i: (0, 0)),
          pl.BlockSpec((1, H1), lambda i: (0, 0)),
          pl.BlockSpec((H1, H2), lambda i: (0, 0)),
          pl.BlockSpec((1, H2), lambda i: (0, 0)),
      ],
      out_specs=pl.BlockSpec((BB, H2), lambda i: (i, 0)),
      out_shape=jax.ShapeDtypeStruct((B, H2), jnp.float32),
  )(emb128, W1p, b1.reshape(1, H1), W2, b2.reshape(1, H2))


def kernel(x, tables, W1, b1, W2, b2):
  x = x.astype(jnp.int32)
  f = jnp.arange(F, dtype=jnp.int32)
  base = ((f // 8) * VP * 8 + (f % 8))[None, :]
  fidx = x * 8 + base
  fidx = jnp.concatenate(
      [fidx, jnp.zeros((B, FP - F), jnp.int32)], axis=1)
  idx3 = fidx.reshape(B // 8, 8, 4, 8).transpose(0, 2, 1, 3).reshape(
      NW, NCH, CH)
  tab_flat = _tc_relayout(tables)
  emb = _sc_gather(tab_flat, idx3)
  emb128 = emb.reshape(S * D // 128, 128)
  W1p = jnp.concatenate(
      [W1, jnp.zeros((FP * D - F * D, H1), jnp.float32)], axis=0)
  return _mlp(emb128, W1p, b1, W2, b2)


# speedup vs baseline: 1.0948x; 1.0948x over previous
import functools

import jax
import jax.numpy as jnp
from jax import lax
from jax.experimental import pallas as pl
from jax.experimental.pallas import tpu as pltpu
from jax.experimental.pallas import tpu_sc as plsc

B = 16384
F = 26
V = 100000
D = 16
H1 = 128
H2 = 64

NW = 32
FP = 32
S = B * FP
RPW = S // NW
NCH = 8
CH = RPW // NCH

VC = 6400
NJ = -(-V // VC)
VP = NJ * VC
NB = -(-F // 8)


def _tc_relayout(tables):
  tab_t = jnp.transpose(tables, (0, 2, 1)).reshape(F * D, V)

  def body(i_ref, o_ref):
    o_ref[...] = i_ref[...].T

  lin128 = pl.pallas_call(
      body,
      grid=(NB, NJ),
      in_specs=[pl.BlockSpec((128, VC), lambda nb, j: (nb, j))],
      out_specs=pl.BlockSpec((VC, 128), lambda nb, j: (nb * NJ + j, 0)),
      out_shape=jax.ShapeDtypeStruct((NB * VP, 128), jnp.float32),
  )(tab_t)
  return lin128.reshape(NB * VP * 8, D)


def _sc_gather(tab_flat, idx3):
  mesh = plsc.VectorSubcoreMesh(core_axis_name="c", subcore_axis_name="s")

  @functools.partial(
      pl.kernel,
      out_type=jax.ShapeDtypeStruct((S, D), jnp.float32),
      mesh=mesh,
      compiler_params=pltpu.CompilerParams(use_tc_tiling_on_sc=False),
      scratch_types=[
          pltpu.VMEM((NCH, CH), jnp.int32),
          pltpu.VMEM((2, CH, D), jnp.float32),
          pltpu.SemaphoreType.DMA,
          pltpu.SemaphoreType.DMA,
          pltpu.SemaphoreType.DMA,
          pltpu.SemaphoreType.DMA,
      ],
  )
  def body(tab_hbm, idx_hbm, out_hbm, idx_v, rows_v, g0, g1, o0, o1):
    num_s = lax.axis_size("s")
    wid = lax.axis_index("c") * num_s + lax.axis_index("s")
    base = wid * RPW
    gsem = (g0, g1)
    osem = (o0, o1)
    pltpu.sync_copy(idx_hbm.at[wid], idx_v)

    gathers = [None] * NCH
    outs = [None] * NCH
    gathers[0] = pltpu.async_copy(tab_hbm.at[idx_v.at[0]], rows_v.at[0],
                                  gsem[0])
    for i in range(NCH):
      p = i % 2
      if i + 1 < NCH:
        if i >= 1:
          outs[i - 1].wait()
        gathers[i + 1] = pltpu.async_copy(
            tab_hbm.at[idx_v.at[i + 1]], rows_v.at[1 - p], gsem[1 - p])
      gathers[i].wait()
      outs[i] = pltpu.async_copy(
          rows_v.at[p], out_hbm.at[pl.ds(base + i * CH, CH)], osem[p])
    outs[NCH - 2].wait()
    outs[NCH - 1].wait()

  return body(tab_flat, idx3)


def _mlp(emb128, W1p, b1, W2, b2):
  BB = 2048
  GB = BB * FP * D // 128

  def body(e_ref, w1_ref, b1_ref, w2_ref, b2_ref, o_ref):
    a = e_ref[...].reshape(BB, 4, 128)
    h = b1_ref[...]
    for ct in range(4):
      xct = a[:, ct, :]
      h = h + jnp.dot(xct, w1_ref[ct * 128:(ct + 1) * 128, :],
                      preferred_element_type=jnp.float32)
    h = jnp.where(h >= 0, h, 0.01 * h)
    h = jnp.dot(h, w2_ref[...], preferred_element_type=jnp.float32)
    h = h + b2_ref[...]
    o_ref[...] = jnp.where(h >= 0, h, 0.01 * h)

  return pl.pallas_call(
      body,
      grid=(B // BB,),
      in_specs=[
          pl.BlockSpec((GB, 128), lambda i: (i, 0)),
          pl.BlockSpec((FP * D, H1), lambda i: (0, 0)),
          pl.BlockSpec((1, H1), lambda i: (0, 0)),
          pl.BlockSpec((H1, H2), lambda i: (0, 0)),
          pl.BlockSpec((1, H2), lambda i: (0, 0)),
      ],
      out_specs=pl.BlockSpec((BB, H2), lambda i: (i, 0)),
      out_shape=jax.ShapeDtypeStruct((B, H2), jnp.float32),
  )(emb128, W1p, b1.reshape(1, H1), W2, b2.reshape(1, H2))


def kernel(x, tables, W1, b1, W2, b2):
  x = x.astype(jnp.int32)
  f = jnp.arange(F, dtype=jnp.int32)
  base = ((f // 8) * VP * 8 + (f % 8))[None, :]
  fidx = x * 8 + base
  fidx = jnp.concatenate(
      [fidx, jnp.zeros((B, FP - F), jnp.int32)], axis=1)
  idx3 = fidx.reshape(NW, NCH, CH)
  tab_flat = _tc_relayout(tables)
  emb = _sc_gather(tab_flat, idx3)
  emb128 = emb.reshape(S * D // 128, 128)
  W1p = jnp.concatenate(
      [W1, jnp.zeros((FP * D - F * D, H1), jnp.float32)], axis=0)
  return _mlp(emb128, W1p, b1, W2, b2)


# speedup vs baseline: 3.4304x; 3.1334x over previous
import functools

import jax
import jax.numpy as jnp
from jax import lax
from jax.experimental import pallas as pl
from jax.experimental.pallas import tpu as pltpu
from jax.experimental.pallas import tpu_sc as plsc

B = 16384
F = 26
V = 100000
D = 16
H1 = 128
H2 = 64

NW = 32
FP = 32
S = B * FP
RPW = S // NW
NCH = 8
CH = RPW // NCH

VC = 6400
NJ = -(-V // VC)
VP = NJ * VC
NB = -(-F // 8)


def _tc_relayout(tables):
  tab_t = jnp.transpose(tables, (0, 2, 1)).reshape(F * D, V)

  def body(i_ref, o_ref):
    o_ref[...] = i_ref[...].T

  lin128 = pl.pallas_call(
      body,
      grid=(NB, NJ),
      in_specs=[pl.BlockSpec((128, VC), lambda nb, j: (nb, j))],
      out_specs=pl.BlockSpec((VC, 128), lambda nb, j: (nb * NJ + j, 0)),
      out_shape=jax.ShapeDtypeStruct((NB * VP, 128), jnp.float32),
  )(tab_t)
  return lin128.reshape(NB * VP * 8, D)


def _sc_gather(tab_flat, idx3):
  mesh = plsc.VectorSubcoreMesh(core_axis_name="c", subcore_axis_name="s")

  @functools.partial(
      pl.kernel,
      out_type=jax.ShapeDtypeStruct((S, D), jnp.float32),
      mesh=mesh,
      compiler_params=pltpu.CompilerParams(use_tc_tiling_on_sc=False),
      scratch_types=[
          pltpu.VMEM((NCH, CH), jnp.int32),
          pltpu.VMEM((2, CH, D), jnp.float32),
          pltpu.SemaphoreType.DMA,
          pltpu.SemaphoreType.DMA,
          pltpu.SemaphoreType.DMA,
          pltpu.SemaphoreType.DMA,
      ],
  )
  def body(tab_hbm, idx_hbm, out_hbm, idx_v, rows_v, g0, g1, o0, o1):
    num_s = lax.axis_size("s")
    wid = lax.axis_index("c") * num_s + lax.axis_index("s")
    base = wid * RPW
    gsem = (g0, g1)
    osem = (o0, o1)
    pltpu.sync_copy(idx_hbm.at[wid], idx_v)

    gathers = [None] * NCH
    outs = [None] * NCH
    gathers[0] = pltpu.async_copy(tab_hbm.at[idx_v.at[0]], rows_v.at[0],
                                  gsem[0])
    for i in range(NCH):
      p = i % 2
      if i + 1 < NCH:
        if i >= 1:
          outs[i - 1].wait()
        gathers[i + 1] = pltpu.async_copy(
            tab_hbm.at[idx_v.at[i + 1]], rows_v.at[1 - p], gsem[1 - p])
      gathers[i].wait()
      outs[i] = pltpu.async_copy(
          rows_v.at[p], out_hbm.at[pl.ds(base + i * CH, CH)], osem[p])
    outs[NCH - 2].wait()
    outs[NCH - 1].wait()

  return body(tab_flat, idx3)


def _mlp(emb128, W1p, b1, W2, b2):
  BB = 2048
  GB = BB * FP * D // 128

  def body(e_ref, w1_ref, b1_ref, w2_ref, b2_ref, o_ref):
    a = e_ref[...].reshape(BB, 4, 128)
    h = b1_ref[...]
    for ct in range(4):
      xct = a[:, ct, :]
      h = h + jnp.dot(xct, w1_ref[ct * 128:(ct + 1) * 128, :],
                      preferred_element_type=jnp.float32)
    h = jnp.where(h >= 0, h, 0.01 * h)
    h = jnp.dot(h, w2_ref[...], preferred_element_type=jnp.float32)
    h = h + b2_ref[...]
    o_ref[...] = jnp.where(h >= 0, h, 0.01 * h)

  return pl.pallas_call(
      body,
      grid=(B // BB,),
      in_specs=[
          pl.BlockSpec((GB, 128), lambda i: (i, 0)),
          pl.BlockSpec((FP * D, H1), lambda i: (0, 0)),
          pl.BlockSpec((1, H1), lambda i: (0, 0)),
          pl.BlockSpec((H1, H2), lambda i: (0, 0)),
          pl.BlockSpec((1, H2), lambda i: (0, 0)),
      ],
      out_specs=pl.BlockSpec((BB, H2), lambda i: (i, 0)),
      out_shape=jax.ShapeDtypeStruct((B, H2), jnp.float32),
  )(emb128, W1p, b1.reshape(1, H1), W2, b2.reshape(1, H2))


def kernel(x, tables, W1, b1, W2, b2):
  x = x.astype(jnp.int32)
  f = jnp.arange(F, dtype=jnp.int32)
  base = ((f // 8) * VP * 8 + (f % 8))[None, :]
  fidx = x * 8 + base
  fidx = jnp.concatenate([fidx, fidx[:, :FP - F]], axis=1)
  idx3 = fidx.reshape(NW, NCH, CH)
  tab_flat = _tc_relayout(tables)
  emb = _sc_gather(tab_flat, idx3)
  emb128 = emb.reshape(S * D // 128, 128)
  W1p = jnp.concatenate(
      [W1, jnp.zeros((FP * D - F * D, H1), jnp.float32)], axis=0)
  return _mlp(emb128, W1p, b1, W2, b2)


# speedup vs baseline: 3.6065x; 1.0514x over previous
import functools

import jax
import jax.numpy as jnp
from jax import lax
from jax.experimental import pallas as pl
from jax.experimental.pallas import tpu as pltpu
from jax.experimental.pallas import tpu_sc as plsc

B = 16384
F = 26
V = 100000
D = 16
H1 = 128
H2 = 64

NW = 32
FP = 32
S = B * FP
RPW = S // NW
NCH = 8
CH = RPW // NCH

VC = 12800
NJ = -(-V // VC)
VP = NJ * VC
NB = -(-F // 8)


def _tc_relayout(tables):
  tab_t = jnp.transpose(tables, (0, 2, 1)).reshape(F * D, V)

  def body(i_ref, o_ref):
    o_ref[...] = i_ref[...].T

  lin128 = pl.pallas_call(
      body,
      grid=(NB, NJ),
      in_specs=[pl.BlockSpec((128, VC), lambda nb, j: (nb, j))],
      out_specs=pl.BlockSpec((VC, 128), lambda nb, j: (nb * NJ + j, 0)),
      out_shape=jax.ShapeDtypeStruct((NB * VP, 128), jnp.float32),
  )(tab_t)
  return lin128.reshape(NB * VP * 8, D)


def _sc_gather(tab_flat, idx3):
  mesh = plsc.VectorSubcoreMesh(core_axis_name="c", subcore_axis_name="s")

  @functools.partial(
      pl.kernel,
      out_type=jax.ShapeDtypeStruct((S, D), jnp.float32),
      mesh=mesh,
      compiler_params=pltpu.CompilerParams(use_tc_tiling_on_sc=False),
      scratch_types=[
          pltpu.VMEM((NCH, CH), jnp.int32),
          pltpu.VMEM((2, CH, D), jnp.float32),
          pltpu.SemaphoreType.DMA,
          pltpu.SemaphoreType.DMA,
          pltpu.SemaphoreType.DMA,
          pltpu.SemaphoreType.DMA,
      ],
  )
  def body(tab_hbm, idx_hbm, out_hbm, idx_v, rows_v, g0, g1, o0, o1):
    num_s = lax.axis_size("s")
    wid = lax.axis_index("c") * num_s + lax.axis_index("s")
    base = wid * RPW
    gsem = (g0, g1)
    osem = (o0, o1)
    pltpu.sync_copy(idx_hbm.at[wid], idx_v)

    gathers = [None] * NCH
    outs = [None] * NCH
    gathers[0] = pltpu.async_copy(tab_hbm.at[idx_v.at[0]], rows_v.at[0],
                                  gsem[0])
    for i in range(NCH):
      p = i % 2
      if i + 1 < NCH:
        if i >= 1:
          outs[i - 1].wait()
        gathers[i + 1] = pltpu.async_copy(
            tab_hbm.at[idx_v.at[i + 1]], rows_v.at[1 - p], gsem[1 - p])
      gathers[i].wait()
      outs[i] = pltpu.async_copy(
          rows_v.at[p], out_hbm.at[pl.ds(base + i * CH, CH)], osem[p])
    outs[NCH - 2].wait()
    outs[NCH - 1].wait()

  return body(tab_flat, idx3)


def _mlp(emb128, W1p, b1, W2, b2):
  BB = 2048
  GB = BB * FP * D // 128

  def body(e_ref, w1_ref, b1_ref, w2_ref, b2_ref, o_ref):
    a = e_ref[...].reshape(BB, 4, 128)
    h = b1_ref[...]
    for ct in range(4):
      xct = a[:, ct, :]
      h = h + jnp.dot(xct, w1_ref[ct * 128:(ct + 1) * 128, :],
                      preferred_element_type=jnp.float32)
    h = jnp.where(h >= 0, h, 0.01 * h)
    h = jnp.dot(h, w2_ref[...], preferred_element_type=jnp.float32)
    h = h + b2_ref[...]
    o_ref[...] = jnp.where(h >= 0, h, 0.01 * h)

  return pl.pallas_call(
      body,
      grid=(B // BB,),
      in_specs=[
          pl.BlockSpec((GB, 128), lambda i: (i, 0)),
          pl.BlockSpec((FP * D, H1), lambda i: (0, 0)),
          pl.BlockSpec((1, H1), lambda i: (0, 0)),
          pl.BlockSpec((H1, H2), lambda i: (0, 0)),
          pl.BlockSpec((1, H2), lambda i: (0, 0)),
      ],
      out_specs=pl.BlockSpec((BB, H2), lambda i: (i, 0)),
      out_shape=jax.ShapeDtypeStruct((B, H2), jnp.float32),
  )(emb128, W1p, b1.reshape(1, H1), W2, b2.reshape(1, H2))


def kernel(x, tables, W1, b1, W2, b2):
  x = x.astype(jnp.int32)
  f = jnp.arange(F, dtype=jnp.int32)
  base = ((f // 8) * VP * 8 + (f % 8))[None, :]
  fidx = x * 8 + base
  fidx = jnp.concatenate([fidx, fidx[:, :FP - F]], axis=1)
  idx3 = fidx.reshape(NW, NCH, CH)
  tab_flat = _tc_relayout(tables)
  emb = _sc_gather(tab_flat, idx3)
  emb128 = emb.reshape(S * D // 128, 128)
  W1p = jnp.concatenate(
      [W1, jnp.zeros((FP * D - F * D, H1), jnp.float32)], axis=0)
  return _mlp(emb128, W1p, b1, W2, b2)


# speedup vs baseline: 3.6688x; 1.0173x over previous
import functools

import jax
import jax.numpy as jnp
from jax import lax
from jax.experimental import pallas as pl
from jax.experimental.pallas import tpu as pltpu
from jax.experimental.pallas import tpu_sc as plsc

B = 16384
F = 26
V = 100000
D = 16
H1 = 128
H2 = 64

NW = 32
FP = 32
S = B * FP
RPW = S // NW
NCH = 8
CH = RPW // NCH

VC = 25600
NJ = -(-V // VC)
VP = NJ * VC
NB = -(-F // 8)


def _tc_relayout(tables):
  tab_t = jnp.transpose(tables, (0, 2, 1)).reshape(F * D, V)

  def body(i_ref, o_ref):
    o_ref[...] = i_ref[...].T

  lin128 = pl.pallas_call(
      body,
      grid=(NB, NJ),
      in_specs=[pl.BlockSpec((128, VC), lambda nb, j: (nb, j))],
      out_specs=pl.BlockSpec((VC, 128), lambda nb, j: (nb * NJ + j, 0)),
      out_shape=jax.ShapeDtypeStruct((NB * VP, 128), jnp.float32),
  )(tab_t)
  return lin128.reshape(NB * VP * 8, D)


def _sc_gather(tab_flat, idx3):
  mesh = plsc.VectorSubcoreMesh(core_axis_name="c", subcore_axis_name="s")

  @functools.partial(
      pl.kernel,
      out_type=jax.ShapeDtypeStruct((S, D), jnp.float32),
      mesh=mesh,
      compiler_params=pltpu.CompilerParams(use_tc_tiling_on_sc=False),
      scratch_types=[
          pltpu.VMEM((NCH, CH), jnp.int32),
          pltpu.VMEM((2, CH, D), jnp.float32),
          pltpu.SemaphoreType.DMA,
          pltpu.SemaphoreType.DMA,
          pltpu.SemaphoreType.DMA,
          pltpu.SemaphoreType.DMA,
      ],
  )
  def body(tab_hbm, idx_hbm, out_hbm, idx_v, rows_v, g0, g1, o0, o1):
    num_s = lax.axis_size("s")
    wid = lax.axis_index("c") * num_s + lax.axis_index("s")
    base = wid * RPW
    gsem = (g0, g1)
    osem = (o0, o1)
    pltpu.sync_copy(idx_hbm.at[wid], idx_v)

    gathers = [None] * NCH
    outs = [None] * NCH
    gathers[0] = pltpu.async_copy(tab_hbm.at[idx_v.at[0]], rows_v.at[0],
                                  gsem[0])
    for i in range(NCH):
      p = i % 2
      if i + 1 < NCH:
        if i >= 1:
          outs[i - 1].wait()
        gathers[i + 1] = pltpu.async_copy(
            tab_hbm.at[idx_v.at[i + 1]], rows_v.at[1 - p], gsem[1 - p])
      gathers[i].wait()
      outs[i] = pltpu.async_copy(
          rows_v.at[p], out_hbm.at[pl.ds(base + i * CH, CH)], osem[p])
    outs[NCH - 2].wait()
    outs[NCH - 1].wait()

  return body(tab_flat, idx3)


def _mlp(emb128, W1p, b1, W2, b2):
  BB = 2048
  GB = BB * FP * D // 128

  def body(e_ref, w1_ref, b1_ref, w2_ref, b2_ref, o_ref):
    a = e_ref[...].reshape(BB, 4, 128)
    h = b1_ref[...]
    for ct in range(4):
      xct = a[:, ct, :]
      h = h + jnp.dot(xct, w1_ref[ct * 128:(ct + 1) * 128, :],
                      preferred_element_type=jnp.float32)
    h = jnp.where(h >= 0, h, 0.01 * h)
    h = jnp.dot(h, w2_ref[...], preferred_element_type=jnp.float32)
    h = h + b2_ref[...]
    o_ref[...] = jnp.where(h >= 0, h, 0.01 * h)

  return pl.pallas_call(
      body,
      grid=(B // BB,),
      in_specs=[
          pl.BlockSpec((GB, 128), lambda i: (i, 0)),
          pl.BlockSpec((FP * D, H1), lambda i: (0, 0)),
          pl.BlockSpec((1, H1), lambda i: (0, 0)),
          pl.BlockSpec((H1, H2), lambda i: (0, 0)),
          pl.BlockSpec((1, H2), lambda i: (0, 0)),
      ],
      out_specs=pl.BlockSpec((BB, H2), lambda i: (i, 0)),
      out_shape=jax.ShapeDtypeStruct((B, H2), jnp.float32),
  )(emb128, W1p, b1.reshape(1, H1), W2, b2.reshape(1, H2))


def kernel(x, tables, W1, b1, W2, b2):
  x = x.astype(jnp.int32)
  f = jnp.arange(F, dtype=jnp.int32)
  base = ((f // 8) * VP * 8 + (f % 8))[None, :]
  fidx = x * 8 + base
  fidx = jnp.concatenate([fidx, fidx[:, :FP - F]], axis=1)
  idx3 = fidx.reshape(NW, NCH, CH)
  tab_flat = _tc_relayout(tables)
  emb = _sc_gather(tab_flat, idx3)
  emb128 = emb.reshape(S * D // 128, 128)
  W1p = jnp.concatenate(
      [W1, jnp.zeros((FP * D - F * D, H1), jnp.float32)], axis=0)
  return _mlp(emb128, W1p, b1, W2, b2)
